# Initial kernel scaffold; baseline (speedup 1.0000x reference)
import jax
import jax.numpy as jnp
from jax.experimental import pallas as pl


def kernel(phi, squid_current, g_table, ib_list):
    raise NotImplementedError("write your pallas kernel here")


# speedup vs baseline: 2142.0762x; 2142.0762x over previous
import jax
import jax.numpy as jnp
from jax.experimental import pallas as pl
from jax.experimental.pallas import tpu as pltpu


def _basis_coeffs(g_table):
    sx = jnp.diff(g_table, axis=1)
    cx = jnp.concatenate([g_table[:, :1], sx[:, :1], jnp.diff(sx, axis=1)], axis=1)
    sy = jnp.diff(cx, axis=0)
    return jnp.concatenate([cx[:1], sy[:1], jnp.diff(sy, axis=0)], axis=0)


def _tc_body(c_ref, pp_ref, phi_ref, sc_ref, o_ref):
    p = phi_ref[...]
    s = sc_ref[...]
    m = p - jnp.floor(p)
    pe = jnp.minimum(m, 1.0 - m)
    x = jnp.minimum(pe * 16.0, 8.0)
    y = jnp.clip((s - pp_ref[0, 0]) * pp_ref[0, 1], 0.0, 4.0)
    xb = [x] + [jnp.maximum(x - float(w), 0.0) for w in range(1, 8)]
    yb = [y] + [jnp.maximum(y - float(h), 0.0) for h in range(1, 4)]
    out = None
    for j in range(5):
        acc = c_ref[j, 0] + c_ref[j, 1] * xb[0]
        for i in range(2, 9):
            acc = acc + c_ref[j, i] * xb[i - 1]
        out = acc if j == 0 else out + yb[j - 1] * acc
    o_ref[...] = out


def kernel(phi, squid_current, g_table, ib_list):
    coeffs = _basis_coeffs(g_table)
    ib_min = ib_list[0]
    yscale = 4.0 / (ib_list[-1] - ib_list[0])
    pp = jnp.stack([ib_min, yscale]).reshape(1, 2)
    n_rows, n_cols = phi.shape
    block_rows = 256
    return pl.pallas_call(
        _tc_body,
        grid=(n_rows // block_rows,),
        in_specs=[
            pl.BlockSpec(memory_space=pltpu.SMEM),
            pl.BlockSpec(memory_space=pltpu.SMEM),
            pl.BlockSpec((block_rows, n_cols), lambda i: (i, 0)),
            pl.BlockSpec((block_rows, n_cols), lambda i: (i, 0)),
        ],
        out_specs=pl.BlockSpec((block_rows, n_cols), lambda i: (i, 0)),
        out_shape=jax.ShapeDtypeStruct((n_rows, n_cols), jnp.float32),
    )(coeffs, pp, phi, squid_current)
